# baseline (speedup 1.0000x reference)
import jax
import jax.numpy as jnp
from jax import lax
from jax.experimental import pallas as pl
from jax.experimental.pallas import tpu as pltpu
from jax.experimental.pallas import tpu_sc as plsc

_N = 5_000_000
_B = 1_000_000

_NC = 2
_NS = 16
_NW = _NC * _NS
_CHUNK = 31_360
_S = 3_920
_NSUB = _CHUNK // _S


def _body(packed_hbm, idx_hbm, out_rows,
          idx_v, rows_a, rows_b, sem_a, sem_b, sem_oa, sem_ob):
    wid = lax.axis_index("s") * _NC + lax.axis_index("c")
    base = lax.min(wid * _CHUNK, _B - _CHUNK)
    pltpu.sync_copy(idx_hbm.at[pl.ds(base, _CHUNK)], idx_v)

    rows = (rows_a, rows_b)
    sems = (sem_a, sem_b)
    osems = (sem_oa, sem_ob)

    def fire(k):
        return pltpu.async_copy(
            packed_hbm.at[idx_v.at[pl.ds(k * _S, _S)]], rows[k % 2], sems[k % 2])

    h = [fire(0)]
    out_h = [None, None]
    for k in range(_NSUB):
        if k + 1 < _NSUB:
            h.append(fire(k + 1))
        h[k].wait()
        if out_h[k % 2] is not None:
            out_h[k % 2].wait()
        out_h[k % 2] = pltpu.async_copy(
            rows[k % 2], out_rows.at[pl.ds(base + k * _S, _S), :], osems[k % 2])
    for oh in out_h:
        if oh is not None:
            oh.wait()


def kernel(src, dst, edge_idxs, timestamps, idx):
    ts_i = lax.bitcast_convert_type(timestamps, jnp.int32)
    packed = jnp.stack([src, dst, edge_idxs, ts_i,
                        src, src, src, src], axis=1)

    call = pl.kernel(
        _body,
        out_type=jax.ShapeDtypeStruct((_B, 8), jnp.int32),
        mesh=plsc.VectorSubcoreMesh(core_axis_name="c", subcore_axis_name="s"),
        compiler_params=pltpu.CompilerParams(use_tc_tiling_on_sc=False),
        scratch_types=[
            pltpu.VMEM((_CHUNK,), jnp.int32),
            pltpu.VMEM((_S, 8), jnp.int32),
            pltpu.VMEM((_S, 8), jnp.int32),
            pltpu.SemaphoreType.DMA,
            pltpu.SemaphoreType.DMA,
            pltpu.SemaphoreType.DMA,
            pltpu.SemaphoreType.DMA,
        ],
    )
    out_rows = call(packed, idx)
    return (out_rows[:, 0], out_rows[:, 1], out_rows[:, 2],
            lax.bitcast_convert_type(out_rows[:, 3], jnp.float32))
